# baseline (speedup 1.0000x reference)
import functools

import jax
import jax.numpy as jnp
from jax import lax
from jax.experimental import pallas as pl
from jax.experimental.pallas import tpu as pltpu
from jax.experimental.pallas import tpu_sc as plsc

NUM_RELS = 100000
EMB_DIM = 512
BATCH = 16384

_info = plsc.get_sparse_core_info()
_NC, _NS = _info.num_cores, _info.num_subcores
NW = _NC * _NS
B_PER_W = BATCH // NW
LAG = 16

_mesh = plsc.VectorSubcoreMesh(core_axis_name="c", subcore_axis_name="s")


@functools.partial(
    pl.kernel,
    mesh=_mesh,
    out_type=jax.ShapeDtypeStruct((BATCH, EMB_DIM), jnp.float32),
    scratch_types=[
        pltpu.VMEM((B_PER_W,), jnp.int32),
        pltpu.SemaphoreType.DMA,
    ],
)
def _gather_kernel(idx_hbm, table_hbm, out_hbm, idx_v, dsem):
    cid = lax.axis_index("c")
    sid = lax.axis_index("s")
    wid = sid * _NC + cid
    base = wid * B_PER_W
    pltpu.sync_copy(idx_hbm.at[pl.ds(base, B_PER_W)], idx_v)

    def body(g, carry):
        vec = idx_v[pl.ds(g * 16, 16)]
        for j in range(16):
            k = vec[j]
            pltpu.async_copy(
                table_hbm.at[k], out_hbm.at[base + g * 16 + j], dsem)

        @pl.when(g >= 1)
        def _():
            for _ in range(16):
                pltpu.make_async_copy(
                    table_hbm.at[0], out_hbm.at[0], dsem).wait()

        return carry

    lax.fori_loop(0, B_PER_W // 16, body, 0)
    for _ in range(16):
        pltpu.make_async_copy(table_hbm.at[0], out_hbm.at[0], dsem).wait()


def kernel(rel_ids, clip_embs):
    return _gather_kernel(rel_ids.astype(jnp.int32), clip_embs)


# speedup vs baseline: 23.5547x; 23.5547x over previous
import functools

import jax
import jax.numpy as jnp
from jax import lax
from jax.experimental import pallas as pl
from jax.experimental.pallas import tpu as pltpu
from jax.experimental.pallas import tpu_sc as plsc

NUM_RELS = 100000
EMB_DIM = 512
BATCH = 16384

_info = plsc.get_sparse_core_info()
_NC, _NS = _info.num_cores, _info.num_subcores
NW = _NC * _NS
B_PER_W = BATCH // NW
CHUNK = 64
NCHUNK = B_PER_W // CHUNK
NBUF = 3

_mesh = plsc.VectorSubcoreMesh(core_axis_name="c", subcore_axis_name="s")


@functools.partial(
    pl.kernel,
    mesh=_mesh,
    out_type=jax.ShapeDtypeStruct((BATCH, EMB_DIM), jnp.float32),
    scratch_types=[
        pltpu.VMEM((NCHUNK, CHUNK), jnp.int32),
        pltpu.VMEM((NBUF, CHUNK, EMB_DIM), jnp.float32),
    ] + [pltpu.SemaphoreType.DMA] * (2 * NBUF + 1),
)
def _gather_kernel(idx_hbm, table_hbm, out_hbm, idx_v, rows_v, *sems):
    gsem = sems[:NBUF]
    wsem = sems[NBUF:2 * NBUF]
    isem = sems[2 * NBUF]
    wid = lax.axis_index("s") * _NC + lax.axis_index("c")
    base = wid * B_PER_W
    ih = [pltpu.async_copy(
        idx_hbm.at[pl.ds(base + j * CHUNK, CHUNK)], idx_v.at[j], isem)
        for j in range(NCHUNK)]
    for h in ih[:NBUF]:
        h.wait()

    def gather(j):
        b = j % NBUF
        return pltpu.async_copy(table_hbm.at[idx_v.at[j]], rows_v.at[b],
                                gsem[b])

    def writeback(j):
        b = j % NBUF
        return pltpu.async_copy(
            rows_v.at[b], out_hbm.at[pl.ds(base + j * CHUNK, CHUNK)], wsem[b])

    gh = {}
    wh = {}
    for b in range(NBUF):
        gh[b] = gather(b)
    for h in ih[NBUF:]:
        h.wait()
    for j in range(NCHUNK):
        pj = j - 1
        if pj >= 0 and pj + NBUF < NCHUNK:
            wh[pj].wait()
            gh[pj + NBUF] = gather(pj + NBUF)
        gh[j].wait()
        wh[j] = writeback(j)
    for j in range(NCHUNK - NBUF, NCHUNK):
        wh[j].wait()


def kernel(rel_ids, clip_embs):
    return _gather_kernel(rel_ids.astype(jnp.int32), clip_embs)


# speedup vs baseline: 23.9809x; 1.0181x over previous
import functools

import jax
import jax.numpy as jnp
from jax import lax
from jax.experimental import pallas as pl
from jax.experimental.pallas import tpu as pltpu
from jax.experimental.pallas import tpu_sc as plsc

NUM_RELS = 100000
EMB_DIM = 512
BATCH = 16384

_info = plsc.get_sparse_core_info()
_NC, _NS = _info.num_cores, _info.num_subcores
NW = _NC * _NS
B_PER_W = BATCH // NW
CHUNK = 32
NCHUNK = B_PER_W // CHUNK
NBUF = 6
DEFER = 2

_mesh = plsc.VectorSubcoreMesh(core_axis_name="c", subcore_axis_name="s")


@functools.partial(
    pl.kernel,
    mesh=_mesh,
    out_type=jax.ShapeDtypeStruct((BATCH, EMB_DIM), jnp.float32),
    scratch_types=[
        pltpu.VMEM((NCHUNK, CHUNK), jnp.int32),
        pltpu.VMEM((NBUF, CHUNK, EMB_DIM), jnp.float32),
    ] + [pltpu.SemaphoreType.DMA] * (2 * NBUF + 1),
)
def _gather_kernel(idx_hbm, table_hbm, out_hbm, idx_v, rows_v, *sems):
    gsem = sems[:NBUF]
    wsem = sems[NBUF:2 * NBUF]
    isem = sems[2 * NBUF]
    wid = lax.axis_index("s") * _NC + lax.axis_index("c")
    base = wid * B_PER_W
    ih = [pltpu.async_copy(
        idx_hbm.at[pl.ds(base + j * CHUNK, CHUNK)], idx_v.at[j], isem)
        for j in range(NCHUNK)]
    for h in ih[:NBUF]:
        h.wait()

    def gather(j):
        b = j % NBUF
        return pltpu.async_copy(table_hbm.at[idx_v.at[j]], rows_v.at[b],
                                gsem[b])

    def writeback(j):
        b = j % NBUF
        return pltpu.async_copy(
            rows_v.at[b], out_hbm.at[pl.ds(base + j * CHUNK, CHUNK)], wsem[b])

    gh = {}
    wh = {}
    for b in range(NBUF):
        gh[b] = gather(b)
    for h in ih[NBUF:]:
        h.wait()
    for j in range(NCHUNK):
        pj = j - DEFER
        if pj >= 0 and pj + NBUF < NCHUNK:
            wh[pj].wait()
            gh[pj + NBUF] = gather(pj + NBUF)
        gh[j].wait()
        wh[j] = writeback(j)
    for j in range(NCHUNK):
        if j + DEFER >= NCHUNK or j + NBUF >= NCHUNK:
            wh[j].wait()


def kernel(rel_ids, clip_embs):
    return _gather_kernel(rel_ids.astype(jnp.int32), clip_embs)
